# baseline (speedup 1.0000x reference)
import functools

import jax
import jax.numpy as jnp
from jax import lax
from jax.experimental import pallas as pl
from jax.experimental.pallas import tpu as pltpu
from jax.experimental.pallas import tpu_sc as plsc

V = 100000
B = 1024
D = 64
NNEG = 5
KBT = 2048
KSTEPS = (V + KBT - 1) // KBT

_NC = 2
_NS = 16
_NW = _NC * _NS
_ROWS = (NNEG + 1) * B
_RPW = _ROWS // _NW


def _matmul_body(inv_ref, l_ref, g_ref, ct_ref, out_ref):
    k = pl.program_id(0)
    rem = V - k * KBT
    w = l_ref[...] + g_ref[...] * inv_ref[...]
    rowmask = lax.broadcasted_iota(jnp.int32, (KBT, B), 0) < rem
    w = jnp.where(rowmask, w, 0.0)
    lanemask = lax.broadcasted_iota(jnp.int32, (D, KBT), 1) < rem
    ct = jnp.where(lanemask, ct_ref[...], 0.0)

    @pl.when(k == 0)
    def _():
        out_ref[...] = jnp.zeros_like(out_ref)

    out_ref[...] += jnp.dot(
        ct.astype(jnp.bfloat16), w.astype(jnp.bfloat16),
        preferred_element_type=jnp.float32,
    )


def _score_body(g_ref, r_ref, p_ref, o_ref):
    rep2 = jnp.concatenate([r_ref[...], r_ref[...]], axis=1)
    lane = lax.broadcasted_iota(jnp.int32, (_ROWS, 2 * D), 1)
    half = (lane // D) == p_ref[...]
    d = jnp.sum(jnp.where(half, g_ref[...] * rep2, 0.0), axis=1, keepdims=True)
    row = lax.broadcasted_iota(jnp.int32, (_ROWS, 1), 0)
    x = jnp.where(row % 6 == 0, -d, d)
    sp = jnp.maximum(x, 0.0) + jnp.log1p(jnp.exp(-jnp.abs(x)))
    o_ref[0, 0] = jnp.sum(sp) * (1.0 / B)


@functools.cache
def _make_gather():
    @functools.partial(
        pl.kernel,
        mesh=plsc.VectorSubcoreMesh(core_axis_name="c", subcore_axis_name="s"),
        out_type=jax.ShapeDtypeStruct((_ROWS, 2 * D), jnp.float32),
        scratch_types=[
            pltpu.VMEM((_RPW,), jnp.int32),
            pltpu.VMEM((_RPW, 2 * D), jnp.float32),
            pltpu.SemaphoreType.DMA,
        ],
        compiler_params=pltpu.CompilerParams(use_tc_tiling_on_sc=False),
    )
    def _gather_rows(idx_hbm, table_hbm, out_hbm, idx_v, rows_v, sem):
        wid = lax.axis_index("s") * _NC + lax.axis_index("c")
        base = wid * _RPW
        pltpu.sync_copy(idx_hbm.at[pl.ds(base, _RPW)], idx_v)
        pltpu.async_copy(table_hbm.at[idx_v], rows_v, sem).wait()
        pltpu.sync_copy(rows_v, out_hbm.at[pl.ds(base, _RPW)])

    return _gather_rows


def kernel(center_w, local_context_w, global_context_w, negative_ws, lengths, center_emb, context_emb):
    invT = (1.0 / lengths).T
    emb_vT = pl.pallas_call(
        _matmul_body,
        grid=(KSTEPS,),
        in_specs=[
            pl.BlockSpec((1, B), lambda k: (0, 0)),
            pl.BlockSpec((KBT, B), lambda k: (k, 0)),
            pl.BlockSpec((KBT, B), lambda k: (k, 0)),
            pl.BlockSpec((D, KBT), lambda k: (0, k)),
        ],
        out_specs=pl.BlockSpec((D, B), lambda k: (0, 0)),
        out_shape=jax.ShapeDtypeStruct((D, B), jnp.float32),
    )(invT, local_context_w.T, global_context_w.T, context_emb.T)

    idx = jnp.concatenate([center_w[:, None], negative_ws], axis=1)
    idx = idx.reshape(-1).astype(jnp.int32)
    par = (idx & 1)[:, None]
    idx2 = (idx >> 1) + (emb_vT[0, 0] * 0.0).astype(jnp.int32)

    table2 = jnp.reshape(center_emb, (V // 2, 2 * D))
    gathered = _make_gather()(idx2, table2)

    rep6 = jnp.repeat(emb_vT.T, NNEG + 1, axis=0)

    out = pl.pallas_call(
        _score_body,
        in_specs=[
            pl.BlockSpec((_ROWS, 2 * D), lambda: (0, 0)),
            pl.BlockSpec((_ROWS, D), lambda: (0, 0)),
            pl.BlockSpec((_ROWS, 1), lambda: (0, 0)),
        ],
        out_specs=pl.BlockSpec(memory_space=pltpu.SMEM),
        out_shape=jax.ShapeDtypeStruct((1, 1), jnp.float32),
    )(gathered, rep6, par)

    return out[0, 0]


# speedup vs baseline: 1.1642x; 1.1642x over previous
import functools

import jax
import jax.numpy as jnp
from jax import lax
from jax.experimental import pallas as pl
from jax.experimental.pallas import tpu as pltpu
from jax.experimental.pallas import tpu_sc as plsc

V = 100000
B = 1024
D = 64
NNEG = 5
KBT = 2048
KSTEPS = (V + KBT - 1) // KBT
HP = KBT // 2
VP = KSTEPS * HP

_NC = 2
_NS = 16
_NW = _NC * _NS
_ROWS = (NNEG + 1) * B
_RPW = _ROWS // _NW


def _matmul_body(inv_ref, l_ref, g_ref, ct_ref, ce_ref, out_ref, tab_ref):
    k = pl.program_id(0)
    rem = V - k * KBT
    w = l_ref[...] + g_ref[...] * inv_ref[...]
    rowmask = lax.broadcasted_iota(jnp.int32, (KBT, B), 0) < rem
    w = jnp.where(rowmask, w, 0.0)
    lanemask = lax.broadcasted_iota(jnp.int32, (D, KBT), 1) < rem
    ct = jnp.where(lanemask, ct_ref[...], 0.0)

    @pl.when(k == 0)
    def _():
        out_ref[...] = jnp.zeros_like(out_ref)

    out_ref[...] += jnp.dot(
        ct.astype(jnp.bfloat16), w.astype(jnp.bfloat16),
        preferred_element_type=jnp.float32,
    )

    ce = lax.transpose(ce_ref[...], (1, 0))
    tab_ref[...] = jnp.concatenate([ce[:HP, :], ce[HP:, :]], axis=1)


def _score_body(g_ref, evt_ref, p_ref, o_ref):
    emb_v = lax.transpose(evt_ref[...], (1, 0))
    rep = jnp.concatenate([emb_v] * (NNEG + 1), axis=0)
    rep2 = jnp.concatenate([rep, rep], axis=1)
    lane = lax.broadcasted_iota(jnp.int32, (_ROWS, 2 * D), 1)
    half = (lane // D) == p_ref[...]
    d = jnp.sum(jnp.where(half, g_ref[...] * rep2, 0.0), axis=1, keepdims=True)
    row = lax.broadcasted_iota(jnp.int32, (_ROWS, 1), 0)
    x = jnp.where(row < B, -d, d)
    sp = jnp.maximum(x, 0.0) + jnp.log1p(jnp.exp(-jnp.abs(x)))
    o_ref[0, 0] = jnp.sum(sp) * (1.0 / B)


@functools.cache
def _make_gather():
    @functools.partial(
        pl.kernel,
        mesh=plsc.VectorSubcoreMesh(core_axis_name="c", subcore_axis_name="s"),
        out_type=jax.ShapeDtypeStruct((_ROWS, 2 * D), jnp.float32),
        scratch_types=[
            pltpu.VMEM((_RPW,), jnp.int32),
            pltpu.VMEM((_RPW, 2 * D), jnp.float32),
            pltpu.SemaphoreType.DMA,
        ],
        compiler_params=pltpu.CompilerParams(use_tc_tiling_on_sc=False),
    )
    def _gather_rows(idx_hbm, table_hbm, out_hbm, idx_v, rows_v, sem):
        wid = lax.axis_index("s") * _NC + lax.axis_index("c")
        base = wid * _RPW
        pltpu.sync_copy(idx_hbm.at[pl.ds(base, _RPW)], idx_v)
        pltpu.async_copy(table_hbm.at[idx_v], rows_v, sem).wait()
        pltpu.sync_copy(rows_v, out_hbm.at[pl.ds(base, _RPW)])

    return _gather_rows


def kernel(center_w, local_context_w, global_context_w, negative_ws, lengths, center_emb, context_emb):
    invT = (1.0 / lengths).T
    emb_vT, table2 = pl.pallas_call(
        _matmul_body,
        grid=(KSTEPS,),
        in_specs=[
            pl.BlockSpec((1, B), lambda k: (0, 0)),
            pl.BlockSpec((KBT, B), lambda k: (k, 0)),
            pl.BlockSpec((KBT, B), lambda k: (k, 0)),
            pl.BlockSpec((D, KBT), lambda k: (0, k)),
            pl.BlockSpec((D, KBT), lambda k: (0, k)),
        ],
        out_specs=[
            pl.BlockSpec((D, B), lambda k: (0, 0)),
            pl.BlockSpec((HP, 2 * D), lambda k: (k, 0)),
        ],
        out_shape=[
            jax.ShapeDtypeStruct((D, B), jnp.float32),
            jax.ShapeDtypeStruct((VP, 2 * D), jnp.float32),
        ],
    )(invT, local_context_w.T, global_context_w.T, context_emb.T, center_emb.T)

    idx = jnp.concatenate(
        [center_w.astype(jnp.int32), negative_ws.T.reshape(-1).astype(jnp.int32)])
    par = ((idx & (KBT - 1)) // HP)[:, None]
    idx2 = (idx // KBT) * HP + (idx & (HP - 1))

    gathered = _make_gather()(idx2, table2)

    out = pl.pallas_call(
        _score_body,
        in_specs=[
            pl.BlockSpec((_ROWS, 2 * D), lambda: (0, 0)),
            pl.BlockSpec((D, B), lambda: (0, 0)),
            pl.BlockSpec((_ROWS, 1), lambda: (0, 0)),
        ],
        out_specs=pl.BlockSpec(memory_space=pltpu.SMEM),
        out_shape=jax.ShapeDtypeStruct((1, 1), jnp.float32),
    )(gathered, emb_vT, par)

    return out[0, 0]
